# baseline (speedup 1.0000x reference)
import jax
import jax.numpy as jnp
from jax import lax
from jax.experimental import pallas as pl
from jax.experimental.pallas import tpu as pltpu
from jax.experimental.pallas import tpu_sc as plsc

TOTAL = 32768
DIM = 2048
SEGS = 16
LANES = 16

NC = 2
NS = 16
HALF = DIM // NC
CGRP = HALF // LANES
TPW = TOTAL // NS
TCH = 16
NCH = TPW // TCH
NBUF = 4


def _seg_sum_body(
    flat_hbm, ids_hbm, out_hbm,
    ids_v, b0, b1, b2, b3, acc_v, zrow_v, idx_v, shared_acc,
    s0, s1, s2, s3,
):
    cid = lax.axis_index("c")
    sid = lax.axis_index("s")
    c0 = cid * HALF
    t_base = sid * TPW

    bufs = [b0, b1, b2, b3]
    sems = [s0, s1, s2, s3]

    def start(k, buf, sem):
        pltpu.async_copy(
            flat_hbm.at[pl.ds(t_base + k * TCH, TCH), pl.ds(c0, HALF)],
            buf, sem,
        )

    def wait(buf, sem):
        pltpu.make_async_copy(
            flat_hbm.at[pl.ds(0, TCH), pl.ds(c0, HALF)], buf, sem
        ).wait()

    for c in range(NBUF - 1):
        start(c, bufs[c], sems[c])

    pltpu.sync_copy(ids_hbm, ids_v)

    zero = jnp.zeros((LANES,), jnp.float32)
    idx_v[...] = lax.iota(jnp.int32, LANES)

    for s in range(SEGS):
        for g in range(CGRP):
            acc_v[s, pl.ds(g * LANES, LANES)] = zero
    for g in range(CGRP):
        zrow_v[pl.ds(g * LANES, LANES)] = zero

    s_iota = lax.iota(jnp.int32, LANES)
    lo0 = jnp.zeros((LANES,), jnp.int32)
    hi0 = jnp.full((LANES,), TOTAL, jnp.int32)

    def bs_body(_, carry):
        lo, hi = carry
        mid = lax.div(lo + hi, 2)
        midc = jnp.minimum(mid, TOTAL - 1)
        v = plsc.load_gather(ids_v, [midc])
        go = lo < hi
        pred = v < s_iota
        lo2 = jnp.where(jnp.logical_and(go, pred), mid + 1, lo)
        hi2 = jnp.where(jnp.logical_and(go, jnp.logical_not(pred)), mid, hi)
        return lo2, hi2

    lovec, _ = lax.fori_loop(0, 16, bs_body, (lo0, hi0))

    bnd = [
        jnp.sum(jnp.where(s_iota == s, lovec, 0))
        for s in range(SEGS)
    ] + [jnp.int32(TOTAL)]

    def process_fast(seg, buf):
        def cg_body(g, carry, _buf=buf):
            col = g * LANES
            sets = [zero, zero, zero, zero]
            for t in range(TCH):
                st = t % 4
                sets[st] = sets[st] + _buf[t, pl.ds(col, LANES)]
            tot = (sets[0] + sets[1]) + (sets[2] + sets[3])
            sl = pl.ds(col, LANES)
            acc_v[seg, sl] = acc_v[seg, sl] + tot
            return carry

        lax.fori_loop(0, CGRP, cg_body, 0)

    def process_slow(k, buf):
        t0 = t_base + k * TCH
        for s in range(SEGS):
            lo_s = jnp.maximum(bnd[s], t0) - t0
            hi_s = jnp.minimum(bnd[s + 1], t0 + TCH) - t0

            def tok_body(t, carry, _buf=buf, _s=s):
                def cg_body(g, c2, _t=t):
                    sl = pl.ds(g * LANES, LANES)
                    acc_v[_s, sl] = acc_v[_s, sl] + _buf[_t, sl]
                    return c2

                lax.fori_loop(0, CGRP, cg_body, 0)
                return carry

            lax.fori_loop(lo_s, hi_s, tok_body, 0)

    def process(k, buf):
        t0 = t_base + k * TCH
        seg = jnp.sum((lovec <= t0).astype(jnp.int32)) - 1
        crossing = jnp.sum(
            jnp.logical_and(lovec > t0, lovec < t0 + TCH).astype(jnp.int32)
        )
        is_pure = crossing == 0

        @pl.when(is_pure)
        def _():
            process_fast(seg, buf)

        @pl.when(jnp.logical_not(is_pure))
        def _():
            process_slow(k, buf)

    def chunk_body(k4, carry):
        k = NBUF * k4
        for i in range(NBUF):
            ki = k + i
            wait(bufs[i], sems[i])
            process(ki, bufs[i])

            @pl.when(ki + NBUF - 1 < NCH)
            def _(_ki=ki, _i=i):
                start(
                    _ki + NBUF - 1,
                    bufs[(_i + NBUF - 1) % NBUF],
                    sems[(_i + NBUF - 1) % NBUF],
                )
        return carry

    lax.fori_loop(0, NCH // NBUF, chunk_body, 0)

    pltpu.sync_copy(zrow_v, shared_acc.at[sid])
    plsc.subcore_barrier()
    pltpu.sync_copy(acc_v, shared_acc.at[idx_v], add=True)
    plsc.subcore_barrier()
    pltpu.sync_copy(shared_acc.at[sid], out_hbm.at[sid, pl.ds(c0, HALF)])


@jax.jit
def _seg_sum(flat, segment_ids):
    mesh = plsc.VectorSubcoreMesh(core_axis_name="c", subcore_axis_name="s")
    k = pl.kernel(
        _seg_sum_body,
        mesh=mesh,
        out_type=jax.ShapeDtypeStruct((SEGS, DIM), jnp.float32),
        scratch_types=[
            pltpu.VMEM((TOTAL,), jnp.int32),
            pltpu.VMEM((TCH, HALF), jnp.float32),
            pltpu.VMEM((TCH, HALF), jnp.float32),
            pltpu.VMEM((TCH, HALF), jnp.float32),
            pltpu.VMEM((TCH, HALF), jnp.float32),
            pltpu.VMEM((SEGS, HALF), jnp.float32),
            pltpu.VMEM((HALF,), jnp.float32),
            pltpu.VMEM((LANES,), jnp.int32),
            pltpu.VMEM_SHARED((SEGS, HALF), jnp.float32),
            pltpu.SemaphoreType.DMA,
            pltpu.SemaphoreType.DMA,
            pltpu.SemaphoreType.DMA,
            pltpu.SemaphoreType.DMA,
        ],
        compiler_params=pltpu.CompilerParams(
            use_tc_tiling_on_sc=False, needs_layout_passes=False
        ),
    )
    return k(flat, segment_ids)


def kernel(flat, segment_ids):
    return _seg_sum(flat, segment_ids)


# speedup vs baseline: 1.1116x; 1.1116x over previous
import functools

import jax
import jax.numpy as jnp
from jax import lax
from jax.experimental import pallas as pl
from jax.experimental.pallas import tpu as pltpu
from jax.experimental.pallas import tpu_sc as plsc

TOTAL = 32768
DIM = 2048
SEGS = 16
LANES = 16

NC = 2
NS = 16
NW = NC * NS
COLS = DIM // NW
VECS = COLS // LANES
CHUNK = 512
NCHUNK = TOTAL // CHUNK


def _seg_sum_body(
    flat_hbm, ids_hbm, out_hbm, ids_v, buf_v, buf2_v, acc_v, sem_a, sem_b
):
    cid = lax.axis_index("c")
    sid = lax.axis_index("s")
    wid = sid * NC + cid
    c0 = wid * COLS

    pltpu.async_copy(
        flat_hbm.at[pl.ds(0, CHUNK), pl.ds(c0, COLS)], buf_v, sem_a
    )

    pltpu.sync_copy(ids_hbm, ids_v)

    zero = jnp.zeros((LANES,), jnp.float32)
    for s in range(SEGS):
        for j in range(VECS):
            acc_v[s, pl.ds(j * LANES, LANES)] = zero

    s_iota = lax.iota(jnp.int32, LANES)
    lo0 = jnp.zeros((LANES,), jnp.int32)
    hi0 = jnp.full((LANES,), TOTAL, jnp.int32)

    def bs_body(_, carry):
        lo, hi = carry
        mid = lax.div(lo + hi, 2)
        midc = jnp.minimum(mid, TOTAL - 1)
        v = plsc.load_gather(ids_v, [midc])
        go = lo < hi
        pred = v < s_iota
        lo2 = jnp.where(jnp.logical_and(go, pred), mid + 1, lo)
        hi2 = jnp.where(jnp.logical_and(go, jnp.logical_not(pred)), mid, hi)
        return lo2, hi2

    lovec, _ = lax.fori_loop(0, 16, bs_body, (lo0, hi0))

    bnd = [
        jnp.sum(jnp.where(s_iota == s, lovec, 0))
        for s in range(SEGS)
    ] + [jnp.int32(TOTAL)]

    def start(k, buf, sem):
        pltpu.async_copy(
            flat_hbm.at[pl.ds(k * CHUNK, CHUNK), pl.ds(c0, COLS)], buf, sem
        )

    def wait(buf, sem):
        pltpu.make_async_copy(
            flat_hbm.at[pl.ds(0, CHUNK), pl.ds(c0, COLS)], buf, sem
        ).wait()

    def process_slow(k, buf):
        t0 = k * CHUNK
        for s in range(SEGS):
            lo_s = jnp.maximum(bnd[s], t0)
            hi_s = jnp.minimum(bnd[s + 1], t0 + CHUNK)

            def tok_body(t, accs, _t0=t0, _buf=buf):
                r = t - _t0
                return tuple(
                    accs[j] + _buf[r, pl.ds(j * LANES, LANES)]
                    for j in range(VECS)
                )

            accs = lax.fori_loop(
                lo_s, hi_s, tok_body, tuple(zero for _ in range(VECS))
            )
            for j in range(VECS):
                sl = pl.ds(j * LANES, LANES)
                acc_v[s, sl] = acc_v[s, sl] + accs[j]

    UNROLL = 8
    NSETS = 4

    def process_fast(seg, buf):
        def body(i, carry, _buf=buf):
            accs = list(carry)
            r0 = i * UNROLL
            for u in range(UNROLL):
                st = u % NSETS
                for j in range(VECS):
                    idx = st * VECS + j
                    accs[idx] = accs[idx] + _buf[r0 + u, pl.ds(j * LANES, LANES)]
            return tuple(accs)

        accs = lax.fori_loop(
            0, CHUNK // UNROLL, body, tuple(zero for _ in range(NSETS * VECS))
        )
        for j in range(VECS):
            tot = accs[j]
            for st in range(1, NSETS):
                tot = tot + accs[st * VECS + j]
            sl = pl.ds(j * LANES, LANES)
            acc_v[seg, sl] = acc_v[seg, sl] + tot

    def process(k, buf):
        t0 = k * CHUNK
        seg = jnp.sum((lovec <= t0).astype(jnp.int32)) - 1
        crossing = jnp.sum(
            jnp.logical_and(lovec > t0, lovec < t0 + CHUNK).astype(jnp.int32)
        )
        is_pure = crossing == 0

        @pl.when(is_pure)
        def _():
            process_fast(seg, buf)

        @pl.when(jnp.logical_not(is_pure))
        def _():
            process_slow(k, buf)

    def chunk_body(k2, carry):
        k = 2 * k2
        start(k + 1, buf2_v, sem_b)
        wait(buf_v, sem_a)
        process(k, buf_v)

        @pl.when(k + 2 < NCHUNK)
        def _():
            start(k + 2, buf_v, sem_a)

        wait(buf2_v, sem_b)
        process(k + 1, buf2_v)
        return carry

    lax.fori_loop(0, NCHUNK // 2, chunk_body, 0)

    pltpu.sync_copy(acc_v, out_hbm.at[:, pl.ds(c0, COLS)])


@jax.jit
def _seg_sum(flat, segment_ids):
    mesh = plsc.VectorSubcoreMesh(core_axis_name="c", subcore_axis_name="s")
    k = pl.kernel(
        _seg_sum_body,
        mesh=mesh,
        out_type=jax.ShapeDtypeStruct((SEGS, DIM), jnp.float32),
        scratch_types=[
            pltpu.VMEM((TOTAL,), jnp.int32),
            pltpu.VMEM((CHUNK, COLS), jnp.float32),
            pltpu.VMEM((CHUNK, COLS), jnp.float32),
            pltpu.VMEM((SEGS, COLS), jnp.float32),
            pltpu.SemaphoreType.DMA,
            pltpu.SemaphoreType.DMA,
        ],
        compiler_params=pltpu.CompilerParams(
            use_tc_tiling_on_sc=False, needs_layout_passes=False
        ),
    )
    return k(flat, segment_ids)


def kernel(flat, segment_ids):
    return _seg_sum(flat, segment_ids)
